# Initial kernel scaffold; baseline (speedup 1.0000x reference)
import jax
import jax.numpy as jnp
from jax.experimental import pallas as pl


def kernel(x, batch, p1_w1, p1_b1, p1_w2, p1_b2, pf0_w, pf0_b, pf1_w, pf1_b, bn_g, bn_b, l0_w, l0_b, l1_w, l1_b, l2_w, l2_b, l3_w, l3_b, l4_w, l4_b, out_w, out_b):
    raise NotImplementedError("write your pallas kernel here")


# speedup vs baseline: 11.7397x; 11.7397x over previous
import jax
import jax.numpy as jnp
from jax import lax
from jax.experimental import pallas as pl

B = 32
N = 1024
D_IN = 3
W = 128
DIM2 = 768
N_LABELS = 4
_BIG = 3.0e38


def _bdot(a, b):
    return jnp.dot(a.astype(jnp.bfloat16), b.astype(jnp.bfloat16),
                   preferred_element_type=jnp.float32)


def _gather(oh, v):
    return jnp.dot(oh, v, preferred_element_type=jnp.float32,
                   precision=lax.Precision.HIGHEST)


def _pairwise_d2(f):
    sq = jnp.sum(f * f, axis=1, keepdims=True)
    f16 = f.astype(jnp.bfloat16)
    g = lax.dot_general(f16, f16, (((1,), (1,)), ((), ())),
                        preferred_element_type=jnp.float32)
    return sq + jnp.transpose(sq) - 2.0 * g


def _top2_onehots(d2):
    iota = lax.broadcasted_iota(jnp.int32, d2.shape, 1)
    m1 = jnp.min(d2, axis=1, keepdims=True)
    idx1 = jnp.min(jnp.where(d2 == m1, iota, N), axis=1, keepdims=True)
    sel1 = iota == idx1
    d2b = jnp.where(sel1, _BIG, d2)
    m2 = jnp.min(d2b, axis=1, keepdims=True)
    idx2 = jnp.min(jnp.where(d2b == m2, iota, N), axis=1, keepdims=True)
    return sel1.astype(jnp.float32), (iota == idx2).astype(jnp.float32)


def _edge_conv1(f, w1, b1, w2, b2):
    oh1, oh2 = _top2_onehots(_pairwise_d2(f))

    def branch(oh):
        e = jnp.concatenate([f, _gather(oh, f) - f], axis=1)
        z = jax.nn.relu(_bdot(e, w1) + b1)
        return jax.nn.relu(_bdot(z, w2) + b2)

    return jnp.maximum(branch(oh1), branch(oh2))


def _edge_conv_single(f, w, b):
    oh1, oh2 = _top2_onehots(_pairwise_d2(f))

    def branch(oh):
        e = jnp.concatenate([f, _gather(oh, f) - f], axis=1)
        return jax.nn.relu(_bdot(e, w) + b)

    return jnp.maximum(branch(oh1), branch(oh2))


def _graph_kernel(x_ref, p1w1_ref, p1b1_ref, p1w2_ref, p1b2_ref,
                  pf0w_ref, pf0b_ref, pf1w_ref, pf1b_ref, out_ref):
    xg = x_ref[...]
    y1 = _edge_conv1(xg, p1w1_ref[...], p1b1_ref[...],
                     p1w2_ref[...], p1b2_ref[...])
    y2 = _edge_conv_single(y1, pf0w_ref[...], pf0b_ref[...])
    y3 = _edge_conv_single(y2, pf1w_ref[...], pf1b_ref[...])
    for k, yk in enumerate((y1, y2, y3)):
        out_ref[0, 0:1, k * W:(k + 1) * W] = jnp.mean(yk, axis=0, keepdims=True)
        out_ref[0, 0:1, 384 + k * W:384 + (k + 1) * W] = (
            jnp.max(yk, axis=0, keepdims=True))


def _head_kernel(p_ref, bng_ref, bnb_ref,
                 w0_ref, b0_ref, w1_ref, b1_ref, w2_ref, b2_ref,
                 w3_ref, b3_ref, w4_ref, b4_ref, ow_ref, ob_ref, out_ref):
    p = p_ref[...]
    mu = jnp.mean(p, axis=0, keepdims=True)
    var = jnp.mean((p - mu) ** 2, axis=0, keepdims=True)
    h = (p - mu) / jnp.sqrt(var + 1e-5) * bng_ref[...] + bnb_ref[...]
    for w_ref, b_ref in ((w0_ref, b0_ref), (w1_ref, b1_ref), (w2_ref, b2_ref),
                         (w3_ref, b3_ref), (w4_ref, b4_ref)):
        hw = _bdot(h, w_ref[...]) + b_ref[...]
        h = jnp.where(hw >= 0, hw, 0.01 * hw)
    out_ref[...] = _bdot(h, ow_ref[...]) + ob_ref[...]


def kernel(x, batch, p1_w1, p1_b1, p1_w2, p1_b2, pf0_w, pf0_b, pf1_w, pf1_b,
           bn_g, bn_b, l0_w, l0_b, l1_w, l1_b, l2_w, l2_b, l3_w, l3_b,
           l4_w, l4_b, out_w, out_b):
    del batch

    def whole(shape):
        return pl.BlockSpec(shape, lambda b: (0, 0))

    pooled = pl.pallas_call(
        _graph_kernel,
        grid=(B,),
        in_specs=[
            pl.BlockSpec((N, D_IN), lambda b: (b, 0)),
            whole((2 * D_IN, W)), whole((1, W)),
            whole((W, W)), whole((1, W)),
            whole((2 * W, W)), whole((1, W)),
            whole((2 * W, W)), whole((1, W)),
        ],
        out_specs=pl.BlockSpec((1, 1, DIM2), lambda b: (b, 0, 0)),
        out_shape=jax.ShapeDtypeStruct((B, 1, DIM2), jnp.float32),
    )(x, p1_w1, p1_b1.reshape(1, -1), p1_w2, p1_b2.reshape(1, -1),
      pf0_w, pf0_b.reshape(1, -1), pf1_w, pf1_b.reshape(1, -1))
    pooled = pooled.reshape(B, DIM2)

    o = pl.pallas_call(
        _head_kernel,
        out_shape=jax.ShapeDtypeStruct((B, N_LABELS), jnp.float32),
    )(pooled, bn_g.reshape(1, -1), bn_b.reshape(1, -1),
      l0_w, l0_b.reshape(1, -1), l1_w, l1_b.reshape(1, -1),
      l2_w, l2_b.reshape(1, -1), l3_w, l3_b.reshape(1, -1),
      l4_w, l4_b.reshape(1, -1), out_w, out_b.reshape(1, -1))
    return o.reshape(-1)
